# Initial kernel scaffold; baseline (speedup 1.0000x reference)
import jax
import jax.numpy as jnp
from jax.experimental import pallas as pl


def kernel(x, edge_index, edge_features, W_nr, b_nr, W_er, b_er, W1_0, b1_0, W2_0, b2_0, W1_1, b1_1, W2_1, b2_1, eps, W_src, W_dst, w_e):
    raise NotImplementedError("write your pallas kernel here")


# speedup vs baseline: 4.7622x; 4.7622x over previous
import functools

import jax
import jax.numpy as jnp
from jax import lax
from jax.experimental import pallas as pl
from jax.experimental.pallas import tpu as pltpu
from jax.experimental.pallas import tpu_sc as plsc

N = 10000
E = 320000
D = 128
F = 64
DOT = 32
NC, NS, LANES = 2, 16, 16
NW = NC * NS
NPAD = 10240
SUB = 128
SPW = 80
E_PAD = NW * SPW * SUB
NROW = NW * SPW
RPT = NPAD // NS
ZR = 128

@functools.cache
def _mesh():
    return plsc.VectorSubcoreMesh(
        core_axis_name="c", subcore_axis_name="s",
        num_cores=NC, num_subcores=NS)


def _mm_relu_kernel(x_ref, w_ref, b_ref, o_ref):
    o_ref[...] = jnp.maximum(x_ref[...] @ w_ref[...] + b_ref[...], 0.0)


def _mm_relu(x, w, b, br=1280):
    n, k = x.shape
    m = w.shape[1]
    return pl.pallas_call(
        _mm_relu_kernel,
        grid=(n // br,),
        in_specs=[
            pl.BlockSpec((br, k), lambda i: (i, 0)),
            pl.BlockSpec((k, m), lambda i: (0, 0)),
            pl.BlockSpec((1, m), lambda i: (0, 0)),
        ],
        out_specs=pl.BlockSpec((br, m), lambda i: (i, 0)),
        out_shape=jax.ShapeDtypeStruct((n, m), jnp.float32),
    )(x, w, b.reshape(1, -1))


def _gin_mlp_kernel(h_ref, p0_ref, p1_ref, s_ref, w1_ref, b1_ref, w2_ref,
                    b2_ref, o_ref):
    z = h_ref[...] * s_ref[0, 0] + p0_ref[...] + p1_ref[...]
    t = jnp.maximum(z @ w1_ref[...] + b1_ref[...], 0.0)
    o_ref[...] = jnp.maximum(t @ w2_ref[...] + b2_ref[...], 0.0)


def _gin_mlp(h, p0, p1, scale, w1, b1, w2, b2, br=1280):
    n = h.shape[0]
    return pl.pallas_call(
        _gin_mlp_kernel,
        grid=(n // br,),
        in_specs=[
            pl.BlockSpec((br, F), lambda i: (i, 0)),
            pl.BlockSpec((br, F), lambda i: (i, 0)),
            pl.BlockSpec((br, F), lambda i: (i, 0)),
            pl.BlockSpec((1, 1), lambda i: (0, 0)),
            pl.BlockSpec((F, F), lambda i: (0, 0)),
            pl.BlockSpec((1, F), lambda i: (0, 0)),
            pl.BlockSpec((F, F), lambda i: (0, 0)),
            pl.BlockSpec((1, F), lambda i: (0, 0)),
        ],
        out_specs=pl.BlockSpec((br, F), lambda i: (i, 0)),
        out_shape=jax.ShapeDtypeStruct((n, F), jnp.float32),
    )(h, p0, p1, scale.reshape(1, 1), w1, b1.reshape(1, -1), w2,
      b2.reshape(1, -1))


def _gin_mlp_proj_kernel(h_ref, p0_ref, p1_ref, s_ref, w1_ref, b1_ref, w2_ref,
                         b2_ref, ws_ref, wd_ref, ps_ref, pd_ref):
    z = h_ref[...] * s_ref[0, 0] + p0_ref[...] + p1_ref[...]
    t = jnp.maximum(z @ w1_ref[...] + b1_ref[...], 0.0)
    h2 = jnp.maximum(t @ w2_ref[...] + b2_ref[...], 0.0)
    ps_ref[...] = h2 @ ws_ref[...]
    pd_ref[...] = h2 @ wd_ref[...]


def _gin_mlp_proj(h, p0, p1, scale, w1, b1, w2, b2, ws, wd, br=1280):
    n = h.shape[0]
    return pl.pallas_call(
        _gin_mlp_proj_kernel,
        grid=(n // br,),
        in_specs=[
            pl.BlockSpec((br, F), lambda i: (i, 0)),
            pl.BlockSpec((br, F), lambda i: (i, 0)),
            pl.BlockSpec((br, F), lambda i: (i, 0)),
            pl.BlockSpec((1, 1), lambda i: (0, 0)),
            pl.BlockSpec((F, F), lambda i: (0, 0)),
            pl.BlockSpec((1, F), lambda i: (0, 0)),
            pl.BlockSpec((F, F), lambda i: (0, 0)),
            pl.BlockSpec((1, F), lambda i: (0, 0)),
            pl.BlockSpec((F, DOT), lambda i: (0, 0)),
            pl.BlockSpec((F, DOT), lambda i: (0, 0)),
        ],
        out_specs=[
            pl.BlockSpec((br, DOT), lambda i: (i, 0)),
            pl.BlockSpec((br, DOT), lambda i: (i, 0)),
        ],
        out_shape=[
            jax.ShapeDtypeStruct((n, DOT), jnp.float32),
            jax.ShapeDtypeStruct((n, DOT), jnp.float32),
        ],
    )(h, p0, p1, scale.reshape(1, 1), w1, b1.reshape(1, -1), w2,
      b2.reshape(1, -1), ws, wd)


def _eterm_kernel(ef_ref, w_ref, b_ref, we_ref, o_ref):
    t = jnp.maximum(ef_ref[...] @ w_ref[...] + b_ref[...], 0.0)
    o_ref[...] = jnp.sum(t * we_ref[...], axis=1, keepdims=True)


def _eterm(ef, w, b, we, br=1280):
    n = ef.shape[0]
    return pl.pallas_call(
        _eterm_kernel,
        grid=(n // br,),
        in_specs=[
            pl.BlockSpec((br, D), lambda i: (i, 0)),
            pl.BlockSpec((D, F), lambda i: (0, 0)),
            pl.BlockSpec((1, F), lambda i: (0, 0)),
            pl.BlockSpec((1, F), lambda i: (0, 0)),
        ],
        out_specs=pl.BlockSpec((br, 1), lambda i: (i, 0)),
        out_shape=jax.ShapeDtypeStruct((n, 1), jnp.float32),
    )(ef, w, b.reshape(1, -1), we.reshape(1, -1))


def _sc_segment_sum(h_pad, src2d, dst2d):
    @functools.partial(
        pl.kernel,
        out_type=jax.ShapeDtypeStruct((NC, NPAD, F), jnp.float32),
        mesh=_mesh(),
        compiler_params=pltpu.CompilerParams(use_tc_tiling_on_sc=False),
        scratch_types=[
            pltpu.VMEM((SPW, SUB), jnp.int32),
            pltpu.VMEM((SPW, SUB), jnp.int32),
            pltpu.VMEM((SUB, F), jnp.float32),
            pltpu.VMEM((SUB, F), jnp.float32),
            pltpu.VMEM((ZR, F), jnp.float32),
            pltpu.VMEM_SHARED((NPAD, F), jnp.float32),
            pltpu.SemaphoreType.DMA,
            pltpu.SemaphoreType.DMA,
        ],
    )
    def seg_kernel(h_hbm, src_hbm, dst_hbm, out_hbm,
                   sidx, didx, rows_a, rows_b, zbuf, acc, sem_a, sem_b):
        cid = lax.axis_index("c")
        sid = lax.axis_index("s")
        wid = sid * NC + cid

        zv = jnp.zeros((LANES,), jnp.float32)

        def _zrow(i, carry):
            for j in range(F // LANES):
                zbuf[i, pl.ds(j * LANES, LANES)] = zv
            return carry

        lax.fori_loop(0, ZR, _zrow, 0)
        for r in range(RPT // ZR):
            pltpu.sync_copy(zbuf, acc.at[pl.ds(sid * RPT + r * ZR, ZR)])
        plsc.subcore_barrier()

        r0 = wid * SPW
        pltpu.sync_copy(src_hbm.at[pl.ds(r0, SPW)], sidx)
        pltpu.sync_copy(dst_hbm.at[pl.ds(r0, SPW)], didx)

        def _body(g, carry):
            j0 = 2 * g
            j1 = 2 * g + 1
            pltpu.async_copy(h_hbm.at[sidx.at[j0]], rows_a, sem_a)
            pltpu.async_copy(h_hbm.at[sidx.at[j1]], rows_b, sem_b)
            pltpu.make_async_copy(h_hbm.at[sidx.at[j0]], rows_a, sem_a).wait()
            pltpu.sync_copy(rows_a, acc.at[didx.at[j0]], add=True)
            pltpu.make_async_copy(h_hbm.at[sidx.at[j1]], rows_b, sem_b).wait()
            pltpu.sync_copy(rows_b, acc.at[didx.at[j1]], add=True)
            return carry

        lax.fori_loop(0, SPW // 2, _body, 0)

        plsc.subcore_barrier()
        pltpu.sync_copy(acc.at[pl.ds(sid * RPT, RPT)],
                        out_hbm.at[cid, pl.ds(sid * RPT, RPT)])

    return seg_kernel(h_pad, src2d, dst2d)


def _sc_edge_dot(ps_pad, pd_pad, src2d, dst2d, et2d):
    @functools.partial(
        pl.kernel,
        out_type=jax.ShapeDtypeStruct((NROW, SUB), jnp.float32),
        mesh=_mesh(),
        compiler_params=pltpu.CompilerParams(
            use_tc_tiling_on_sc=False, needs_layout_passes=False),
        scratch_types=[
            pltpu.VMEM((SPW, SUB), jnp.int32),
            pltpu.VMEM((SPW, SUB), jnp.int32),
            pltpu.VMEM((SUB, DOT), jnp.float32),
            pltpu.VMEM((SUB, DOT), jnp.float32),
            pltpu.VMEM((SPW, SUB), jnp.float32),
            pltpu.VMEM((SPW, SUB), jnp.float32),
            pltpu.SemaphoreType.DMA,
            pltpu.SemaphoreType.DMA,
        ],
    )
    def dot_kernel(ps_hbm, pd_hbm, src_hbm, dst_hbm, et_hbm, out_hbm,
                   sidx, didx, rows_s, rows_d, ev, ov, sem_a, sem_b):
        cid = lax.axis_index("c")
        sid = lax.axis_index("s")
        wid = sid * NC + cid
        r0 = wid * SPW
        pltpu.sync_copy(src_hbm.at[pl.ds(r0, SPW)], sidx)
        pltpu.sync_copy(dst_hbm.at[pl.ds(r0, SPW)], didx)
        pltpu.sync_copy(et_hbm.at[pl.ds(r0, SPW)], ev)

        def _chunk(j, carry):
            pltpu.async_copy(ps_hbm.at[sidx.at[j]], rows_s, sem_a)
            pltpu.async_copy(pd_hbm.at[didx.at[j]], rows_d, sem_b)
            pltpu.make_async_copy(ps_hbm.at[sidx.at[j]], rows_s, sem_a).wait()
            pltpu.make_async_copy(pd_hbm.at[didx.at[j]], rows_d, sem_b).wait()
            for g in range(SUB // LANES):
                eids = lax.iota(jnp.int32, LANES) + g * LANES
                accv = ev[j, pl.ds(g * LANES, LANES)]
                for t in range(DOT):
                    tt = jnp.full((LANES,), t, jnp.int32)
                    sv = plsc.load_gather(rows_s, [eids, tt])
                    dv = plsc.load_gather(rows_d, [eids, tt])
                    accv = accv + sv * dv
                ov[j, pl.ds(g * LANES, LANES)] = accv
            return carry

        lax.fori_loop(0, SPW, _chunk, 0)
        pltpu.sync_copy(ov, out_hbm.at[pl.ds(r0, SPW)])

    return dot_kernel(ps_pad, pd_pad, src2d, dst2d, et2d)


def kernel(x, edge_index, edge_features, W_nr, b_nr, W_er, b_er,
           W1_0, b1_0, W2_0, b2_0, W1_1, b1_1, W2_1, b2_1,
           eps, W_src, W_dst, w_e):
    src = edge_index[0]
    dst = edge_index[1]
    extra = NPAD - N
    pad_idx = (N + (jnp.arange(E_PAD - E, dtype=jnp.int32) % extra))
    src2d = jnp.concatenate([src, pad_idx]).reshape(NROW, SUB)
    dst2d = jnp.concatenate([dst, pad_idx]).reshape(NROW, SUB)

    x_pad = jnp.pad(x, ((0, extra), (0, 0)))
    h0 = _mm_relu(x_pad, W_nr, b_nr)
    parts0 = _sc_segment_sum(h0, src2d, dst2d)
    h1 = _gin_mlp(h0, parts0[0], parts0[1], 1.0 + eps[0],
                  W1_0, b1_0, W2_0, b2_0)
    parts1 = _sc_segment_sum(h1, src2d, dst2d)
    ps, pd = _gin_mlp_proj(h1, parts1[0], parts1[1], 1.0 + eps[1],
                           W1_1, b1_1, W2_1, b2_1, W_src, W_dst)
    et = _eterm(edge_features, W_er, b_er, w_e)
    et2d = jnp.concatenate(
        [et.reshape(-1), jnp.zeros((E_PAD - E,), jnp.float32)]
    ).reshape(NROW, SUB)
    out2d = _sc_edge_dot(ps, pd, src2d, dst2d, et2d)
    return out2d.reshape(-1)[:E]


# speedup vs baseline: 5.7259x; 1.2024x over previous
import functools

import jax
import jax.numpy as jnp
from jax import lax
from jax.experimental import pallas as pl
from jax.experimental.pallas import tpu as pltpu
from jax.experimental.pallas import tpu_sc as plsc

N = 10000
E = 320000
D = 128
F = 64
DOT = 32
NC, NS, LANES = 2, 16, 16
NW = NC * NS
NPAD = 10240
SUB = 128
SPW = 80
E_PAD = NW * SPW * SUB
NROW = NW * SPW
RPT = NPAD // NS
ZR = 128

@functools.cache
def _mesh():
    return plsc.VectorSubcoreMesh(
        core_axis_name="c", subcore_axis_name="s",
        num_cores=NC, num_subcores=NS)


def _mm_relu_kernel(x_ref, w_ref, b_ref, o_ref):
    o_ref[...] = jnp.maximum(x_ref[...] @ w_ref[...] + b_ref[...], 0.0)


def _mm_relu(x, w, b, br=1280):
    n, k = x.shape
    m = w.shape[1]
    return pl.pallas_call(
        _mm_relu_kernel,
        grid=(n // br,),
        in_specs=[
            pl.BlockSpec((br, k), lambda i: (i, 0)),
            pl.BlockSpec((k, m), lambda i: (0, 0)),
            pl.BlockSpec((1, m), lambda i: (0, 0)),
        ],
        out_specs=pl.BlockSpec((br, m), lambda i: (i, 0)),
        out_shape=jax.ShapeDtypeStruct((n, m), jnp.float32),
    )(x, w, b.reshape(1, -1))


def _gin_mlp_kernel(h_ref, p_ref, s_ref, w1_ref, b1_ref, w2_ref,
                    b2_ref, o_ref):
    z = h_ref[...] * s_ref[0, 0] + p_ref[0] + p_ref[1]
    t = jnp.maximum(z @ w1_ref[...] + b1_ref[...], 0.0)
    o_ref[...] = jnp.maximum(t @ w2_ref[...] + b2_ref[...], 0.0)


def _gin_mlp(h, parts, scale, w1, b1, w2, b2, br=1280):
    n = h.shape[0]
    return pl.pallas_call(
        _gin_mlp_kernel,
        grid=(n // br,),
        in_specs=[
            pl.BlockSpec((br, F), lambda i: (i, 0)),
            pl.BlockSpec((NC, br, F), lambda i: (0, i, 0)),
            pl.BlockSpec((1, 1), lambda i: (0, 0)),
            pl.BlockSpec((F, F), lambda i: (0, 0)),
            pl.BlockSpec((1, F), lambda i: (0, 0)),
            pl.BlockSpec((F, F), lambda i: (0, 0)),
            pl.BlockSpec((1, F), lambda i: (0, 0)),
        ],
        out_specs=pl.BlockSpec((br, F), lambda i: (i, 0)),
        out_shape=jax.ShapeDtypeStruct((n, F), jnp.float32),
    )(h, parts, scale.reshape(1, 1), w1, b1.reshape(1, -1), w2,
      b2.reshape(1, -1))


def _gin_mlp_proj_kernel(h_ref, p_ref, s_ref, w1_ref, b1_ref, w2_ref,
                         b2_ref, ws_ref, wd_ref, ps_ref, pd_ref):
    z = h_ref[...] * s_ref[0, 0] + p_ref[0] + p_ref[1]
    t = jnp.maximum(z @ w1_ref[...] + b1_ref[...], 0.0)
    h2 = jnp.maximum(t @ w2_ref[...] + b2_ref[...], 0.0)
    ps_ref[...] = h2 @ ws_ref[...]
    pd_ref[...] = h2 @ wd_ref[...]


def _gin_mlp_proj(h, parts, scale, w1, b1, w2, b2, ws, wd, br=1280):
    n = h.shape[0]
    return pl.pallas_call(
        _gin_mlp_proj_kernel,
        grid=(n // br,),
        in_specs=[
            pl.BlockSpec((br, F), lambda i: (i, 0)),
            pl.BlockSpec((NC, br, F), lambda i: (0, i, 0)),
            pl.BlockSpec((1, 1), lambda i: (0, 0)),
            pl.BlockSpec((F, F), lambda i: (0, 0)),
            pl.BlockSpec((1, F), lambda i: (0, 0)),
            pl.BlockSpec((F, F), lambda i: (0, 0)),
            pl.BlockSpec((1, F), lambda i: (0, 0)),
            pl.BlockSpec((F, DOT), lambda i: (0, 0)),
            pl.BlockSpec((F, DOT), lambda i: (0, 0)),
        ],
        out_specs=[
            pl.BlockSpec((br, DOT), lambda i: (i, 0)),
            pl.BlockSpec((br, DOT), lambda i: (i, 0)),
        ],
        out_shape=[
            jax.ShapeDtypeStruct((n, DOT), jnp.float32),
            jax.ShapeDtypeStruct((n, DOT), jnp.float32),
        ],
    )(h, parts, scale.reshape(1, 1), w1, b1.reshape(1, -1), w2,
      b2.reshape(1, -1), ws, wd)


def _eterm_kernel(ef_ref, w_ref, b_ref, we_ref, o_ref):
    t = jnp.maximum(ef_ref[...] @ w_ref[...] + b_ref[...], 0.0)
    s = jnp.sum(t * we_ref[...], axis=1)
    o_ref[...] = s.reshape(o_ref.shape)


def _eterm(ef, w, b, we, br=1280):
    n = ef.shape[0]
    return pl.pallas_call(
        _eterm_kernel,
        grid=(n // br,),
        in_specs=[
            pl.BlockSpec((br, D), lambda i: (i, 0)),
            pl.BlockSpec((D, F), lambda i: (0, 0)),
            pl.BlockSpec((1, F), lambda i: (0, 0)),
            pl.BlockSpec((1, F), lambda i: (0, 0)),
        ],
        out_specs=pl.BlockSpec((1, br // SUB, SUB), lambda i: (i, 0, 0)),
        out_shape=jax.ShapeDtypeStruct((n // br, br // SUB, SUB), jnp.float32),
    )(ef, w, b.reshape(1, -1), we.reshape(1, -1))


def _sc_segment_sum(h_pad, src2d, dst2d):
    @functools.partial(
        pl.kernel,
        out_type=jax.ShapeDtypeStruct((NC, NPAD, F), jnp.float32),
        mesh=_mesh(),
        compiler_params=pltpu.CompilerParams(use_tc_tiling_on_sc=False),
        scratch_types=[
            pltpu.VMEM((SPW, SUB), jnp.int32),
            pltpu.VMEM((SPW, SUB), jnp.int32),
            pltpu.VMEM((SUB, F), jnp.float32),
            pltpu.VMEM((SUB, F), jnp.float32),
            pltpu.VMEM((ZR, F), jnp.float32),
            pltpu.VMEM_SHARED((NPAD, F), jnp.float32),
            pltpu.SemaphoreType.DMA,
            pltpu.SemaphoreType.DMA,
            pltpu.SemaphoreType.DMA,
            pltpu.SemaphoreType.DMA,
        ],
    )
    def seg_kernel(h_hbm, src_hbm, dst_hbm, out_hbm,
                   sidx, didx, rows_a, rows_b, zbuf, acc,
                   sem_ga, sem_gb, sem_sa, sem_sb):
        cid = lax.axis_index("c")
        sid = lax.axis_index("s")
        wid = sid * NC + cid

        zv = jnp.zeros((LANES,), jnp.float32)

        def _zrow(i, carry):
            for j in range(F // LANES):
                zbuf[i, pl.ds(j * LANES, LANES)] = zv
            return carry

        lax.fori_loop(0, ZR, _zrow, 0)
        for r in range(RPT // ZR):
            pltpu.sync_copy(zbuf, acc.at[pl.ds(sid * RPT + r * ZR, ZR)])
        plsc.subcore_barrier()

        r0 = wid * SPW
        pltpu.sync_copy(src_hbm.at[pl.ds(r0, SPW)], sidx)
        pltpu.sync_copy(dst_hbm.at[pl.ds(r0, SPW)], didx)

        pltpu.async_copy(h_hbm.at[sidx.at[0]], rows_a, sem_ga)
        pltpu.async_copy(h_hbm.at[sidx.at[1]], rows_b, sem_gb)

        def _body(g, carry):
            j0 = 2 * g
            j1 = 2 * g + 1
            pltpu.make_async_copy(h_hbm.at[sidx.at[j0]], rows_a, sem_ga).wait()
            pltpu.async_copy(rows_a, acc.at[didx.at[j0]], sem_sa, add=True)
            pltpu.make_async_copy(h_hbm.at[sidx.at[j1]], rows_b, sem_gb).wait()
            pltpu.async_copy(rows_b, acc.at[didx.at[j1]], sem_sb, add=True)
            pltpu.make_async_copy(rows_a, acc.at[didx.at[j0]], sem_sa).wait()

            @pl.when(j0 + 2 < SPW)
            def _():
                pltpu.async_copy(h_hbm.at[sidx.at[j0 + 2]], rows_a, sem_ga)

            pltpu.make_async_copy(rows_b, acc.at[didx.at[j1]], sem_sb).wait()

            @pl.when(j1 + 2 < SPW)
            def _():
                pltpu.async_copy(h_hbm.at[sidx.at[j1 + 2]], rows_b, sem_gb)

            return carry

        lax.fori_loop(0, SPW // 2, _body, 0)

        plsc.subcore_barrier()
        pltpu.sync_copy(acc.at[pl.ds(sid * RPT, RPT)],
                        out_hbm.at[cid, pl.ds(sid * RPT, RPT)])

    return seg_kernel(h_pad, src2d, dst2d)


def _sc_edge_dot(ps_pad, pd_pad, src2d, dst2d, et2d):
    @functools.partial(
        pl.kernel,
        out_type=jax.ShapeDtypeStruct((NROW, SUB), jnp.float32),
        mesh=_mesh(),
        compiler_params=pltpu.CompilerParams(
            use_tc_tiling_on_sc=False, needs_layout_passes=False),
        scratch_types=[
            pltpu.VMEM((SPW, SUB), jnp.int32),
            pltpu.VMEM((SPW, SUB), jnp.int32),
            pltpu.VMEM((SUB, DOT), jnp.float32),
            pltpu.VMEM((SUB, DOT), jnp.float32),
            pltpu.VMEM((SUB, DOT), jnp.float32),
            pltpu.VMEM((SUB, DOT), jnp.float32),
            pltpu.VMEM((SPW, SUB), jnp.float32),
            pltpu.VMEM((SPW, SUB), jnp.float32),
            pltpu.SemaphoreType.DMA,
            pltpu.SemaphoreType.DMA,
        ],
    )
    def dot_kernel(ps_hbm, pd_hbm, src_hbm, dst_hbm, et_hbm, out_hbm,
                   sidx, didx, rows_sa, rows_da, rows_sb, rows_db,
                   ev, ov, sem_a, sem_b):
        cid = lax.axis_index("c")
        sid = lax.axis_index("s")
        wid = sid * NC + cid
        r0 = wid * SPW
        pltpu.sync_copy(src_hbm.at[pl.ds(r0, SPW)], sidx)
        pltpu.sync_copy(dst_hbm.at[pl.ds(r0, SPW)], didx)
        pltpu.sync_copy(et_hbm.at[pl.ds(r0, SPW)], ev)

        def _compute(j, rows_s, rows_d):
            for g in range(SUB // LANES):
                eids = lax.iota(jnp.int32, LANES) + g * LANES
                accv = ev[j, pl.ds(g * LANES, LANES)]
                for t in range(DOT):
                    tt = jnp.full((LANES,), t, jnp.int32)
                    sv = plsc.load_gather(rows_s, [eids, tt])
                    dv = plsc.load_gather(rows_d, [eids, tt])
                    accv = accv + sv * dv
                ov[j, pl.ds(g * LANES, LANES)] = accv

        pltpu.async_copy(ps_hbm.at[sidx.at[0]], rows_sa, sem_a)
        pltpu.async_copy(pd_hbm.at[didx.at[0]], rows_da, sem_a)

        def _chunk(g, carry):
            j0 = 2 * g
            j1 = 2 * g + 1
            pltpu.async_copy(ps_hbm.at[sidx.at[j1]], rows_sb, sem_b)
            pltpu.async_copy(pd_hbm.at[didx.at[j1]], rows_db, sem_b)
            pltpu.make_async_copy(ps_hbm.at[sidx.at[j0]], rows_sa, sem_a).wait()
            pltpu.make_async_copy(pd_hbm.at[didx.at[j0]], rows_da, sem_a).wait()
            _compute(j0, rows_sa, rows_da)

            @pl.when(j0 + 2 < SPW)
            def _():
                pltpu.async_copy(ps_hbm.at[sidx.at[j0 + 2]], rows_sa, sem_a)
                pltpu.async_copy(pd_hbm.at[didx.at[j0 + 2]], rows_da, sem_a)

            pltpu.make_async_copy(ps_hbm.at[sidx.at[j1]], rows_sb, sem_b).wait()
            pltpu.make_async_copy(pd_hbm.at[didx.at[j1]], rows_db, sem_b).wait()
            _compute(j1, rows_sb, rows_db)
            return carry

        lax.fori_loop(0, SPW // 2, _chunk, 0)
        pltpu.sync_copy(ov, out_hbm.at[pl.ds(r0, SPW)])

    return dot_kernel(ps_pad, pd_pad, src2d, dst2d, et2d)


def kernel(x, edge_index, edge_features, W_nr, b_nr, W_er, b_er,
           W1_0, b1_0, W2_0, b2_0, W1_1, b1_1, W2_1, b2_1,
           eps, W_src, W_dst, w_e):
    src = edge_index[0]
    dst = edge_index[1]
    extra = NPAD - N
    pad_idx = (N + (jnp.arange(E_PAD - E, dtype=jnp.int32) % extra))
    src2d = jnp.concatenate([src, pad_idx]).reshape(NROW, SUB)
    dst2d = jnp.concatenate([dst, pad_idx]).reshape(NROW, SUB)

    x_pad = jnp.pad(x, ((0, extra), (0, 0)))
    h0 = _mm_relu(x_pad, W_nr, b_nr)
    parts0 = _sc_segment_sum(h0, src2d, dst2d)
    h1 = _gin_mlp(h0, parts0, 1.0 + eps[0], W1_0, b1_0, W2_0, b2_0)
    parts1 = _sc_segment_sum(h1, src2d, dst2d)
    ps, pd = _gin_mlp_proj(h1, parts1, 1.0 + eps[1],
                           W1_1, b1_1, W2_1, b2_1, W_src, W_dst)
    et = _eterm(edge_features, W_er, b_er, w_e).reshape(E // SUB, SUB)
    et2d = jnp.concatenate(
        [et, jnp.zeros((NROW - E // SUB, SUB), jnp.float32)])
    out2d = _sc_edge_dot(ps, pd, src2d, dst2d, et2d)
    return out2d.reshape(-1)[:E]


# speedup vs baseline: 8.4546x; 1.4766x over previous
import functools

import jax
import jax.numpy as jnp
from jax import lax
from jax.experimental import pallas as pl
from jax.experimental.pallas import tpu as pltpu
from jax.experimental.pallas import tpu_sc as plsc

N = 10000
E = 320000
D = 128
F = 64
DOT = 32
NC, NS, LANES = 2, 16, 16
NW = NC * NS
NPAD = 10240
SUB = 128
SPW = 80
E_PAD = NW * SPW * SUB
NROW = NW * SPW
RPT = NPAD // NS
ZR = 128

@functools.cache
def _mesh():
    return plsc.VectorSubcoreMesh(
        core_axis_name="c", subcore_axis_name="s",
        num_cores=NC, num_subcores=NS)


def _mm_relu_kernel(x_ref, w_ref, b_ref, o_ref):
    o_ref[...] = jnp.maximum(x_ref[...] @ w_ref[...] + b_ref[...], 0.0)


def _mm_relu(x, w, b, br=1280):
    n, k = x.shape
    m = w.shape[1]
    return pl.pallas_call(
        _mm_relu_kernel,
        grid=(n // br,),
        in_specs=[
            pl.BlockSpec((br, k), lambda i: (i, 0)),
            pl.BlockSpec((k, m), lambda i: (0, 0)),
            pl.BlockSpec((1, m), lambda i: (0, 0)),
        ],
        out_specs=pl.BlockSpec((br, m), lambda i: (i, 0)),
        out_shape=jax.ShapeDtypeStruct((n, m), jnp.float32),
    )(x, w, b.reshape(1, -1))


def _gin_mlp_kernel(h_ref, p_ref, s_ref, w1_ref, b1_ref, w2_ref,
                    b2_ref, o_ref):
    z = h_ref[...] * s_ref[0, 0] + p_ref[0] + p_ref[1]
    t = jnp.maximum(z @ w1_ref[...] + b1_ref[...], 0.0)
    o_ref[...] = jnp.maximum(t @ w2_ref[...] + b2_ref[...], 0.0)


def _gin_mlp(h, parts, scale, w1, b1, w2, b2, br=1280):
    n = h.shape[0]
    return pl.pallas_call(
        _gin_mlp_kernel,
        grid=(n // br,),
        in_specs=[
            pl.BlockSpec((br, F), lambda i: (i, 0)),
            pl.BlockSpec((NC, br, F), lambda i: (0, i, 0)),
            pl.BlockSpec((1, 1), lambda i: (0, 0)),
            pl.BlockSpec((F, F), lambda i: (0, 0)),
            pl.BlockSpec((1, F), lambda i: (0, 0)),
            pl.BlockSpec((F, F), lambda i: (0, 0)),
            pl.BlockSpec((1, F), lambda i: (0, 0)),
        ],
        out_specs=pl.BlockSpec((br, F), lambda i: (i, 0)),
        out_shape=jax.ShapeDtypeStruct((n, F), jnp.float32),
    )(h, parts, scale.reshape(1, 1), w1, b1.reshape(1, -1), w2,
      b2.reshape(1, -1))


def _gin_mlp_proj_kernel(h_ref, p_ref, s_ref, w1_ref, b1_ref, w2_ref,
                         b2_ref, ws_ref, wd_ref, ps_ref, pd_ref):
    z = h_ref[...] * s_ref[0, 0] + p_ref[0] + p_ref[1]
    t = jnp.maximum(z @ w1_ref[...] + b1_ref[...], 0.0)
    h2 = jnp.maximum(t @ w2_ref[...] + b2_ref[...], 0.0)
    ps_ref[...] = h2 @ ws_ref[...]
    pd_ref[...] = h2 @ wd_ref[...]


def _gin_mlp_proj(h, parts, scale, w1, b1, w2, b2, ws, wd, br=1280):
    n = h.shape[0]
    return pl.pallas_call(
        _gin_mlp_proj_kernel,
        grid=(n // br,),
        in_specs=[
            pl.BlockSpec((br, F), lambda i: (i, 0)),
            pl.BlockSpec((NC, br, F), lambda i: (0, i, 0)),
            pl.BlockSpec((1, 1), lambda i: (0, 0)),
            pl.BlockSpec((F, F), lambda i: (0, 0)),
            pl.BlockSpec((1, F), lambda i: (0, 0)),
            pl.BlockSpec((F, F), lambda i: (0, 0)),
            pl.BlockSpec((1, F), lambda i: (0, 0)),
            pl.BlockSpec((F, DOT), lambda i: (0, 0)),
            pl.BlockSpec((F, DOT), lambda i: (0, 0)),
        ],
        out_specs=[
            pl.BlockSpec((br, DOT), lambda i: (i, 0)),
            pl.BlockSpec((br, DOT), lambda i: (i, 0)),
        ],
        out_shape=[
            jax.ShapeDtypeStruct((n, DOT), jnp.float32),
            jax.ShapeDtypeStruct((n, DOT), jnp.float32),
        ],
    )(h, parts, scale.reshape(1, 1), w1, b1.reshape(1, -1), w2,
      b2.reshape(1, -1), ws, wd)


def _eterm_kernel(ef_ref, w_ref, b_ref, we_ref, o_ref):
    t = jnp.maximum(ef_ref[...] @ w_ref[...] + b_ref[...], 0.0)
    s = jnp.sum(t * we_ref[...], axis=1)
    o_ref[...] = s.reshape(o_ref.shape)


def _eterm(ef, w, b, we, br=1280):
    n = ef.shape[0]
    return pl.pallas_call(
        _eterm_kernel,
        grid=(n // br,),
        in_specs=[
            pl.BlockSpec((br, D), lambda i: (i, 0)),
            pl.BlockSpec((D, F), lambda i: (0, 0)),
            pl.BlockSpec((1, F), lambda i: (0, 0)),
            pl.BlockSpec((1, F), lambda i: (0, 0)),
        ],
        out_specs=pl.BlockSpec((1, br // SUB, SUB), lambda i: (i, 0, 0)),
        out_shape=jax.ShapeDtypeStruct((n // br, br // SUB, SUB), jnp.float32),
    )(ef, w, b.reshape(1, -1), we.reshape(1, -1))


def _sc_segment_sum(h_pad, src2d, dst2d):
    @functools.partial(
        pl.kernel,
        out_type=jax.ShapeDtypeStruct((NC, NPAD, F), jnp.float32),
        mesh=_mesh(),
        compiler_params=pltpu.CompilerParams(use_tc_tiling_on_sc=False),
        scratch_types=[
            pltpu.VMEM((SPW, SUB), jnp.int32),
            pltpu.VMEM((SPW, SUB), jnp.int32),
            pltpu.VMEM((SUB, F), jnp.float32),
            pltpu.VMEM((SUB, F), jnp.float32),
            pltpu.VMEM((ZR, F), jnp.float32),
            pltpu.VMEM_SHARED((NPAD, F), jnp.float32),
            pltpu.SemaphoreType.DMA,
            pltpu.SemaphoreType.DMA,
            pltpu.SemaphoreType.DMA,
            pltpu.SemaphoreType.DMA,
        ],
    )
    def seg_kernel(h_hbm, src_hbm, dst_hbm, out_hbm,
                   sidx, didx, rows_a, rows_b, zbuf, acc,
                   sem_ga, sem_gb, sem_sa, sem_sb):
        cid = lax.axis_index("c")
        sid = lax.axis_index("s")
        wid = sid * NC + cid

        zv = jnp.zeros((LANES,), jnp.float32)

        def _zrow(i, carry):
            for j in range(F // LANES):
                zbuf[i, pl.ds(j * LANES, LANES)] = zv
            return carry

        lax.fori_loop(0, ZR, _zrow, 0)
        for r in range(RPT // ZR):
            pltpu.sync_copy(zbuf, acc.at[pl.ds(sid * RPT + r * ZR, ZR)])
        plsc.subcore_barrier()

        r0 = wid * SPW
        pltpu.sync_copy(src_hbm.at[pl.ds(r0, SPW)], sidx)
        pltpu.sync_copy(dst_hbm.at[pl.ds(r0, SPW)], didx)

        pltpu.async_copy(h_hbm.at[sidx.at[0]], rows_a, sem_ga)
        pltpu.async_copy(h_hbm.at[sidx.at[1]], rows_b, sem_gb)

        def _body(g, carry):
            j0 = 2 * g
            j1 = 2 * g + 1
            pltpu.make_async_copy(h_hbm.at[sidx.at[j0]], rows_a, sem_ga).wait()
            pltpu.async_copy(rows_a, acc.at[didx.at[j0]], sem_sa, add=True)
            pltpu.make_async_copy(h_hbm.at[sidx.at[j1]], rows_b, sem_gb).wait()
            pltpu.async_copy(rows_b, acc.at[didx.at[j1]], sem_sb, add=True)
            pltpu.make_async_copy(rows_a, acc.at[didx.at[j0]], sem_sa).wait()

            @pl.when(j0 + 2 < SPW)
            def _():
                pltpu.async_copy(h_hbm.at[sidx.at[j0 + 2]], rows_a, sem_ga)

            pltpu.make_async_copy(rows_b, acc.at[didx.at[j1]], sem_sb).wait()

            @pl.when(j1 + 2 < SPW)
            def _():
                pltpu.async_copy(h_hbm.at[sidx.at[j1 + 2]], rows_b, sem_gb)

            return carry

        lax.fori_loop(0, SPW // 2, _body, 0)

        plsc.subcore_barrier()
        pltpu.sync_copy(acc.at[pl.ds(sid * RPT, RPT)],
                        out_hbm.at[cid, pl.ds(sid * RPT, RPT)])

    return seg_kernel(h_pad, src2d, dst2d)


def _sc_edge_dot(ps_pad, pd_pad, src2d, dst2d, et2d):
    @functools.partial(
        pl.kernel,
        out_type=jax.ShapeDtypeStruct((NROW, SUB), jnp.float32),
        mesh=_mesh(),
        compiler_params=pltpu.CompilerParams(
            use_tc_tiling_on_sc=False, needs_layout_passes=False),
        scratch_types=[
            pltpu.VMEM((SPW, SUB), jnp.int32),
            pltpu.VMEM((SPW, SUB), jnp.int32),
            pltpu.VMEM((SUB, DOT), jnp.float32),
            pltpu.VMEM((SUB, DOT), jnp.float32),
            pltpu.VMEM((SUB, DOT), jnp.float32),
            pltpu.VMEM((SUB, DOT), jnp.float32),
            pltpu.VMEM((SPW, SUB), jnp.float32),
            pltpu.VMEM((SPW, SUB), jnp.float32),
            pltpu.SemaphoreType.DMA,
            pltpu.SemaphoreType.DMA,
        ],
    )
    def dot_kernel(ps_hbm, pd_hbm, src_hbm, dst_hbm, et_hbm, out_hbm,
                   sidx, didx, rows_sa, rows_da, rows_sb, rows_db,
                   ev, ov, sem_a, sem_b):
        cid = lax.axis_index("c")
        sid = lax.axis_index("s")
        wid = sid * NC + cid
        r0 = wid * SPW
        pltpu.sync_copy(src_hbm.at[pl.ds(r0, SPW)], sidx)
        pltpu.sync_copy(dst_hbm.at[pl.ds(r0, SPW)], didx)
        pltpu.sync_copy(et_hbm.at[pl.ds(r0, SPW)], ev)

        def _compute(j, rows_s, rows_d):
            lanes = lax.iota(jnp.int32, LANES)
            for g in range(SUB // LANES):
                eids = lanes + g * LANES
                accv = ev[j, pl.ds(g * LANES, LANES)]
                for t in range(DOT):
                    tt = (lanes + t) & (DOT - 1)
                    sv = plsc.load_gather(rows_s, [eids, tt])
                    dv = plsc.load_gather(rows_d, [eids, tt])
                    accv = accv + sv * dv
                ov[j, pl.ds(g * LANES, LANES)] = accv

        pltpu.async_copy(ps_hbm.at[sidx.at[0]], rows_sa, sem_a)
        pltpu.async_copy(pd_hbm.at[didx.at[0]], rows_da, sem_a)

        def _chunk(g, carry):
            j0 = 2 * g
            j1 = 2 * g + 1
            pltpu.async_copy(ps_hbm.at[sidx.at[j1]], rows_sb, sem_b)
            pltpu.async_copy(pd_hbm.at[didx.at[j1]], rows_db, sem_b)
            pltpu.make_async_copy(ps_hbm.at[sidx.at[j0]], rows_sa, sem_a).wait()
            pltpu.make_async_copy(pd_hbm.at[didx.at[j0]], rows_da, sem_a).wait()
            _compute(j0, rows_sa, rows_da)

            @pl.when(j0 + 2 < SPW)
            def _():
                pltpu.async_copy(ps_hbm.at[sidx.at[j0 + 2]], rows_sa, sem_a)
                pltpu.async_copy(pd_hbm.at[didx.at[j0 + 2]], rows_da, sem_a)

            pltpu.make_async_copy(ps_hbm.at[sidx.at[j1]], rows_sb, sem_b).wait()
            pltpu.make_async_copy(pd_hbm.at[didx.at[j1]], rows_db, sem_b).wait()
            _compute(j1, rows_sb, rows_db)
            return carry

        lax.fori_loop(0, SPW // 2, _chunk, 0)
        pltpu.sync_copy(ov, out_hbm.at[pl.ds(r0, SPW)])

    return dot_kernel(ps_pad, pd_pad, src2d, dst2d, et2d)


def kernel(x, edge_index, edge_features, W_nr, b_nr, W_er, b_er,
           W1_0, b1_0, W2_0, b2_0, W1_1, b1_1, W2_1, b2_1,
           eps, W_src, W_dst, w_e):
    src = edge_index[0]
    dst = edge_index[1]
    extra = NPAD - N
    pad_idx = (N + (jnp.arange(E_PAD - E, dtype=jnp.int32) % extra))
    src2d = jnp.concatenate([src, pad_idx]).reshape(NROW, SUB)
    dst2d = jnp.concatenate([dst, pad_idx]).reshape(NROW, SUB)

    x_pad = jnp.pad(x, ((0, extra), (0, 0)))
    h0 = _mm_relu(x_pad, W_nr, b_nr)
    parts0 = _sc_segment_sum(h0, src2d, dst2d)
    h1 = _gin_mlp(h0, parts0, 1.0 + eps[0], W1_0, b1_0, W2_0, b2_0)
    parts1 = _sc_segment_sum(h1, src2d, dst2d)
    ps, pd = _gin_mlp_proj(h1, parts1, 1.0 + eps[1],
                           W1_1, b1_1, W2_1, b2_1, W_src, W_dst)
    et = _eterm(edge_features, W_er, b_er, w_e).reshape(E // SUB, SUB)
    et2d = jnp.concatenate(
        [et, jnp.zeros((NROW - E // SUB, SUB), jnp.float32)])
    out2d = _sc_edge_dot(ps, pd, src2d, dst2d, et2d)
    return out2d.reshape(-1)[:E]


# speedup vs baseline: 8.8506x; 1.0468x over previous
import functools

import jax
import jax.numpy as jnp
import numpy as np
from jax import lax
from jax.experimental import pallas as pl
from jax.experimental.pallas import tpu as pltpu
from jax.experimental.pallas import tpu_sc as plsc

N = 10000
E = 320000
D = 128
F = 64
DOT = 32
NC, NS, LANES = 2, 16, 16
NW = NC * NS
NPAD = 10240
SUB = 128
SPW = 80
E_PAD = NW * SPW * SUB
NROW = NW * SPW
RPT = NPAD // NS
ZR = 128

@functools.cache
def _mesh():
    return plsc.VectorSubcoreMesh(
        core_axis_name="c", subcore_axis_name="s",
        num_cores=NC, num_subcores=NS)


def _mm_relu_kernel(x_ref, w_ref, b_ref, o_ref):
    o_ref[...] = jnp.maximum(x_ref[...] @ w_ref[...] + b_ref[...], 0.0)


def _mm_relu(x, w, b, br=1280):
    n, k = x.shape
    m = w.shape[1]
    return pl.pallas_call(
        _mm_relu_kernel,
        grid=(n // br,),
        in_specs=[
            pl.BlockSpec((br, k), lambda i: (i, 0)),
            pl.BlockSpec((k, m), lambda i: (0, 0)),
            pl.BlockSpec((1, m), lambda i: (0, 0)),
        ],
        out_specs=pl.BlockSpec((br, m), lambda i: (i, 0)),
        out_shape=jax.ShapeDtypeStruct((n, m), jnp.float32),
    )(x, w, b.reshape(1, -1))


def _gin_mlp_kernel(h_ref, p_ref, s_ref, w1_ref, b1_ref, w2_ref,
                    b2_ref, o_ref):
    z = h_ref[...] * s_ref[0, 0] + p_ref[0] + p_ref[1]
    t = jnp.maximum(z @ w1_ref[...] + b1_ref[...], 0.0)
    o_ref[...] = jnp.maximum(t @ w2_ref[...] + b2_ref[...], 0.0)


def _gin_mlp(h, parts, scale, w1, b1, w2, b2, br=1280):
    n = h.shape[0]
    return pl.pallas_call(
        _gin_mlp_kernel,
        grid=(n // br,),
        in_specs=[
            pl.BlockSpec((br, F), lambda i: (i, 0)),
            pl.BlockSpec((NC, br, F), lambda i: (0, i, 0)),
            pl.BlockSpec((1, 1), lambda i: (0, 0)),
            pl.BlockSpec((F, F), lambda i: (0, 0)),
            pl.BlockSpec((1, F), lambda i: (0, 0)),
            pl.BlockSpec((F, F), lambda i: (0, 0)),
            pl.BlockSpec((1, F), lambda i: (0, 0)),
        ],
        out_specs=pl.BlockSpec((br, F), lambda i: (i, 0)),
        out_shape=jax.ShapeDtypeStruct((n, F), jnp.float32),
    )(h, parts, scale.reshape(1, 1), w1, b1.reshape(1, -1), w2,
      b2.reshape(1, -1))


def _gin_mlp_proj_kernel(h_ref, p_ref, s_ref, w1_ref, b1_ref, w2_ref,
                         b2_ref, ws_ref, wd_ref, ps_ref, pd_ref):
    z = h_ref[...] * s_ref[0, 0] + p_ref[0] + p_ref[1]
    t = jnp.maximum(z @ w1_ref[...] + b1_ref[...], 0.0)
    h2 = jnp.maximum(t @ w2_ref[...] + b2_ref[...], 0.0)
    ps_ref[...] = h2 @ ws_ref[...]
    pd_ref[...] = h2 @ wd_ref[...]


def _gin_mlp_proj(h, parts, scale, w1, b1, w2, b2, ws, wd, br=1280):
    n = h.shape[0]
    return pl.pallas_call(
        _gin_mlp_proj_kernel,
        grid=(n // br,),
        in_specs=[
            pl.BlockSpec((br, F), lambda i: (i, 0)),
            pl.BlockSpec((NC, br, F), lambda i: (0, i, 0)),
            pl.BlockSpec((1, 1), lambda i: (0, 0)),
            pl.BlockSpec((F, F), lambda i: (0, 0)),
            pl.BlockSpec((1, F), lambda i: (0, 0)),
            pl.BlockSpec((F, F), lambda i: (0, 0)),
            pl.BlockSpec((1, F), lambda i: (0, 0)),
            pl.BlockSpec((F, DOT), lambda i: (0, 0)),
            pl.BlockSpec((F, DOT), lambda i: (0, 0)),
        ],
        out_specs=[
            pl.BlockSpec((br, DOT), lambda i: (i, 0)),
            pl.BlockSpec((br, DOT), lambda i: (i, 0)),
        ],
        out_shape=[
            jax.ShapeDtypeStruct((n, DOT), jnp.float32),
            jax.ShapeDtypeStruct((n, DOT), jnp.float32),
        ],
    )(h, parts, scale.reshape(1, 1), w1, b1.reshape(1, -1), w2,
      b2.reshape(1, -1), ws, wd)


def _eterm_kernel(ef_ref, w_ref, b_ref, we_ref, o_ref):
    t = jnp.maximum(ef_ref[...] @ w_ref[...] + b_ref[...], 0.0)
    s = jnp.sum(t * we_ref[...], axis=1)
    o_ref[...] = s.reshape(o_ref.shape)


def _eterm(ef, w, b, we, blk0, nblk, br=1280):
    return pl.pallas_call(
        _eterm_kernel,
        grid=(nblk,),
        in_specs=[
            pl.BlockSpec((br, D), lambda i: (i + blk0, 0)),
            pl.BlockSpec((D, F), lambda i: (0, 0)),
            pl.BlockSpec((1, F), lambda i: (0, 0)),
            pl.BlockSpec((1, F), lambda i: (0, 0)),
        ],
        out_specs=pl.BlockSpec((1, br // SUB, SUB), lambda i: (i, 0, 0)),
        out_shape=jax.ShapeDtypeStruct((nblk, br // SUB, SUB), jnp.float32),
    )(ef, w, b.reshape(1, -1), we.reshape(1, -1))


def _sc_segment_sum(h_pad, src2d, dst2d):
    @functools.partial(
        pl.kernel,
        out_type=jax.ShapeDtypeStruct((NC, NPAD, F), jnp.float32),
        mesh=_mesh(),
        compiler_params=pltpu.CompilerParams(use_tc_tiling_on_sc=False),
        scratch_types=[
            pltpu.VMEM((SPW, SUB), jnp.int32),
            pltpu.VMEM((SPW, SUB), jnp.int32),
            pltpu.VMEM((SUB, F), jnp.float32),
            pltpu.VMEM((SUB, F), jnp.float32),
            pltpu.VMEM((ZR, F), jnp.float32),
            pltpu.VMEM_SHARED((NPAD, F), jnp.float32),
            pltpu.SemaphoreType.DMA,
            pltpu.SemaphoreType.DMA,
            pltpu.SemaphoreType.DMA,
            pltpu.SemaphoreType.DMA,
        ],
    )
    def seg_kernel(h_hbm, src_hbm, dst_hbm, out_hbm,
                   sidx, didx, rows_a, rows_b, zbuf, acc,
                   sem_ga, sem_gb, sem_sa, sem_sb):
        cid = lax.axis_index("c")
        sid = lax.axis_index("s")
        wid = sid * NC + cid

        zv = jnp.zeros((LANES,), jnp.float32)

        def _zrow(i, carry):
            for j in range(F // LANES):
                zbuf[i, pl.ds(j * LANES, LANES)] = zv
            return carry

        lax.fori_loop(0, ZR, _zrow, 0)
        for r in range(RPT // ZR):
            pltpu.sync_copy(zbuf, acc.at[pl.ds(sid * RPT + r * ZR, ZR)])
        plsc.subcore_barrier()

        r0 = wid * SPW
        pltpu.sync_copy(src_hbm.at[pl.ds(r0, SPW)], sidx)
        pltpu.sync_copy(dst_hbm.at[pl.ds(r0, SPW)], didx)

        pltpu.async_copy(h_hbm.at[sidx.at[0]], rows_a, sem_ga)
        pltpu.async_copy(h_hbm.at[sidx.at[1]], rows_b, sem_gb)

        def _body(g, carry):
            j0 = 2 * g
            j1 = 2 * g + 1
            pltpu.make_async_copy(h_hbm.at[sidx.at[j0]], rows_a, sem_ga).wait()
            pltpu.async_copy(rows_a, acc.at[didx.at[j0]], sem_sa, add=True)
            pltpu.make_async_copy(h_hbm.at[sidx.at[j1]], rows_b, sem_gb).wait()
            pltpu.async_copy(rows_b, acc.at[didx.at[j1]], sem_sb, add=True)
            pltpu.make_async_copy(rows_a, acc.at[didx.at[j0]], sem_sa).wait()

            @pl.when(j0 + 2 < SPW)
            def _():
                pltpu.async_copy(h_hbm.at[sidx.at[j0 + 2]], rows_a, sem_ga)

            pltpu.make_async_copy(rows_b, acc.at[didx.at[j1]], sem_sb).wait()

            @pl.when(j1 + 2 < SPW)
            def _():
                pltpu.async_copy(h_hbm.at[sidx.at[j1 + 2]], rows_b, sem_gb)

            return carry

        lax.fori_loop(0, SPW // 2, _body, 0)

        plsc.subcore_barrier()
        pltpu.sync_copy(acc.at[pl.ds(sid * RPT, RPT)],
                        out_hbm.at[cid, pl.ds(sid * RPT, RPT)])

    return seg_kernel(h_pad, src2d, dst2d)


def _sc_edge_dot(ps_pad, pd_pad, src2d, dst2d):
    @functools.partial(
        pl.kernel,
        out_type=jax.ShapeDtypeStruct((NROW, SUB), jnp.float32),
        mesh=_mesh(),
        compiler_params=pltpu.CompilerParams(
            use_tc_tiling_on_sc=False, needs_layout_passes=False),
        scratch_types=[
            pltpu.VMEM((SPW, SUB), jnp.int32),
            pltpu.VMEM((SPW, SUB), jnp.int32),
            pltpu.VMEM((SUB, DOT), jnp.float32),
            pltpu.VMEM((SUB, DOT), jnp.float32),
            pltpu.VMEM((SUB, DOT), jnp.float32),
            pltpu.VMEM((SUB, DOT), jnp.float32),
            pltpu.VMEM((SPW, SUB), jnp.float32),
            pltpu.SemaphoreType.DMA,
            pltpu.SemaphoreType.DMA,
        ],
    )
    def dot_kernel(ps_hbm, pd_hbm, src_hbm, dst_hbm, out_hbm,
                   sidx, didx, rows_sa, rows_da, rows_sb, rows_db,
                   ov, sem_a, sem_b):
        cid = lax.axis_index("c")
        sid = lax.axis_index("s")
        wid = sid * NC + cid
        r0 = wid * SPW
        pltpu.sync_copy(src_hbm.at[pl.ds(r0, SPW)], sidx)
        pltpu.sync_copy(dst_hbm.at[pl.ds(r0, SPW)], didx)

        def _compute(j, rows_s, rows_d):
            lanes = lax.iota(jnp.int32, LANES)
            for g in range(SUB // LANES):
                eids = lanes + g * LANES
                accv = jnp.zeros((LANES,), jnp.float32)
                for t in range(DOT):
                    tt = (lanes + t) & (DOT - 1)
                    sv = plsc.load_gather(rows_s, [eids, tt])
                    dv = plsc.load_gather(rows_d, [eids, tt])
                    accv = accv + sv * dv
                ov[j, pl.ds(g * LANES, LANES)] = accv

        pltpu.async_copy(ps_hbm.at[sidx.at[0]], rows_sa, sem_a)
        pltpu.async_copy(pd_hbm.at[didx.at[0]], rows_da, sem_a)

        def _chunk(g, carry):
            j0 = 2 * g
            j1 = 2 * g + 1
            pltpu.async_copy(ps_hbm.at[sidx.at[j1]], rows_sb, sem_b)
            pltpu.async_copy(pd_hbm.at[didx.at[j1]], rows_db, sem_b)
            pltpu.make_async_copy(ps_hbm.at[sidx.at[j0]], rows_sa, sem_a).wait()
            pltpu.make_async_copy(pd_hbm.at[didx.at[j0]], rows_da, sem_a).wait()
            _compute(j0, rows_sa, rows_da)

            @pl.when(j0 + 2 < SPW)
            def _():
                pltpu.async_copy(ps_hbm.at[sidx.at[j0 + 2]], rows_sa, sem_a)
                pltpu.async_copy(pd_hbm.at[didx.at[j0 + 2]], rows_da, sem_a)

            pltpu.make_async_copy(ps_hbm.at[sidx.at[j1]], rows_sb, sem_b).wait()
            pltpu.make_async_copy(pd_hbm.at[didx.at[j1]], rows_db, sem_b).wait()
            _compute(j1, rows_sb, rows_db)
            return carry

        lax.fori_loop(0, SPW // 2, _chunk, 0)
        pltpu.sync_copy(ov, out_hbm.at[pl.ds(r0, SPW)])

    return dot_kernel(ps_pad, pd_pad, src2d, dst2d)


def kernel(x, edge_index, edge_features, W_nr, b_nr, W_er, b_er,
           W1_0, b1_0, W2_0, b2_0, W1_1, b1_1, W2_1, b2_1,
           eps, W_src, W_dst, w_e):
    src = edge_index[0]
    dst = edge_index[1]
    extra = NPAD - N
    pad_idx = jnp.asarray(
        N + (np.arange(E_PAD - E, dtype=np.int32) % extra), jnp.int32)
    src2d = jnp.concatenate([src, pad_idx]).reshape(NROW, SUB)
    dst2d = jnp.concatenate([dst, pad_idx]).reshape(NROW, SUB)

    nb = E // 1280
    nb0, nb1 = 83, 83
    nb2 = nb - nb0 - nb1
    et_a = _eterm(edge_features, W_er, b_er, w_e, 0, nb0)
    et_b = _eterm(edge_features, W_er, b_er, w_e, nb0, nb1)
    et_c = _eterm(edge_features, W_er, b_er, w_e, nb0 + nb1, nb2)

    x_pad = jnp.pad(x, ((0, extra), (0, 0)))
    h0 = _mm_relu(x_pad, W_nr, b_nr)
    parts0 = _sc_segment_sum(h0, src2d, dst2d)
    h1 = _gin_mlp(h0, parts0, 1.0 + eps[0], W1_0, b1_0, W2_0, b2_0)
    parts1 = _sc_segment_sum(h1, src2d, dst2d)
    ps, pd = _gin_mlp_proj(h1, parts1, 1.0 + eps[1],
                           W1_1, b1_1, W2_1, b2_1, W_src, W_dst)
    out2d = _sc_edge_dot(ps, pd, src2d, dst2d)
    et = jnp.concatenate([et_a.reshape(-1), et_b.reshape(-1),
                          et_c.reshape(-1)])
    return out2d.reshape(-1)[:E] + et
